# Initial kernel scaffold; baseline (speedup 1.0000x reference)
import jax
import jax.numpy as jnp
from jax.experimental import pallas as pl


def kernel(pred, target):
    raise NotImplementedError("write your pallas kernel here")


# speedup vs baseline: 5.0175x; 5.0175x over previous
import functools

import jax
import jax.numpy as jnp
import numpy as np
from jax import lax
from jax.experimental import pallas as pl
from jax.experimental.pallas import tpu as pltpu
from jax.experimental.pallas import tpu_sc as plsc

ROWS = 128
N = 32768
K = 1638
NC, NS = 2, 16
NW = NC * NS
ROWS_PER_W = ROWS // NW
NCH = N // 16
LN2 = 0.6931471805599453
SQ6_PI = float(np.sqrt(6.0) / np.pi)
SQRT_HALF2 = 1.4142135623730951


def _key_from_val(v):
    b = plsc.bitcast(v, jnp.int32)
    m = lax.shift_right_arithmetic(b, 31)
    return lax.bitwise_xor(b, lax.bitwise_or(m, jnp.int32(-2147483648)))


def _val_from_key(key):
    m = lax.shift_right_arithmetic(key, 31)
    mask = lax.bitwise_or(lax.bitwise_not(m), jnp.int32(-2147483648))
    return plsc.bitcast(lax.bitwise_xor(key, mask), jnp.float32)


def _bsum(v):
    return jnp.full((16,), jnp.sum(v), dtype=v.dtype)


def _sqrt_vec(x):
    bits = plsc.bitcast(x, jnp.int32)
    g = plsc.bitcast(
        lax.shift_right_logical(bits, 1) + jnp.int32(0x1FBD1DF5), jnp.float32)
    for _ in range(4):
        g = 0.5 * (g + x / g)
    return g


def _log_vec(x):
    bits = plsc.bitcast(x, jnp.int32)
    e = lax.bitwise_and(lax.shift_right_logical(bits, 23), jnp.int32(255)) - 127
    m_bits = lax.bitwise_or(
        lax.bitwise_and(bits, jnp.int32(0x007FFFFF)), jnp.int32(0x3F800000))
    m = plsc.bitcast(m_bits, jnp.float32)
    big = m > SQRT_HALF2
    m = jnp.where(big, m * 0.5, m)
    e_f = (e + big.astype(jnp.int32)).astype(jnp.float32)
    s = (m - 1.0) / (m + 1.0)
    s2 = s * s
    p = 2.0 * s * (1.0 + s2 * (1.0 / 3.0 + s2 * (
        0.2 + s2 * (1.0 / 7.0 + s2 * (1.0 / 9.0)))))
    return e_f * LN2 + p


def _body(pred_hbm, target_hbm, out_hbm, dbuf, cand_a, cand_b, hist, accbuf):
    wid = lax.axis_index("s") * NC + lax.axis_index("c")
    lane = lax.broadcasted_iota(jnp.int32, (16,), 0)
    ones_i = jnp.full((16,), 1, dtype=jnp.int32)
    zeros_i = jnp.zeros((16,), dtype=jnp.int32)
    zf = jnp.zeros((16,), dtype=jnp.float32)

    def clear_hist():
        def cl(i, _):
            hist[pl.ds(i * 16, 16)] = zeros_i
            return 0
        lax.fori_loop(0, 256, cl, 0)

    def scan_buckets(kk):
        def tot(b):
            return jnp.sum(hist[pl.ds(b * 16, 16)])

        def cond(st):
            b, csum = st
            return csum + tot(b) < kk

        def body(st):
            b, csum = st
            return b - 1, csum + tot(b)

        return lax.while_loop(cond, body, (jnp.int32(255), jnp.int32(0)))

    def select_topk(data_is_pred, sum_fn):
        clear_hist()

        def hist1(i, _):
            v = dbuf[pl.ds(i * 16, 16)]
            key = _key_from_val(v)
            dig = lax.shift_right_logical(key, 24)
            idx = lax.bitwise_or(lax.shift_left(dig, 4), lane)
            plsc.addupdate_scatter(hist, [idx], ones_i)
            return 0

        lax.fori_loop(0, NCH, hist1, 0)
        sel, csum = scan_buckets(jnp.int32(K))
        kk = K - csum
        c_gt = csum

        def compact1(i, st):
            w, s0, s1 = st
            v = dbuf[pl.ds(i * 16, 16)]
            key = _key_from_val(v)
            dig = lax.shift_right_logical(key, 24)
            m_gt = dig > sel
            m_eq = dig == sel
            s0, s1 = sum_fn(s0, s1, m_gt, v)
            cs = plsc.cumsum(m_eq.astype(jnp.int32))
            plsc.store_scatter(cand_a, [cs + (w - 1)], key, mask=m_eq)
            return w + jnp.sum(m_eq.astype(jnp.int32)), s0, s1

        cnt, s0, s1 = lax.fori_loop(0, NCH, compact1, (jnp.int32(0), zf, zf))

        t_key = lax.shift_left(sel, 24)
        src, dst = cand_a, cand_b
        for sh in (16, 8, 0):
            clear_hist()

            def histn(i, _, src=src, sh=sh, cnt=cnt):
                key = src[pl.ds(i * 16, 16)]
                valid = (i * 16 + lane) < cnt
                dig = lax.bitwise_and(
                    lax.shift_right_logical(key, sh), jnp.int32(255))
                idx = lax.bitwise_or(lax.shift_left(dig, 4), lane)
                plsc.addupdate_scatter(hist, [idx], ones_i, mask=valid)
                return 0

            nch = (cnt + 15) // 16
            lax.fori_loop(0, nch, histn, 0)
            sel, csum = scan_buckets(kk)
            kk = kk - csum
            c_gt = c_gt + csum

            def compactn(i, st, src=src, dst=dst, sh=sh, cnt=cnt, sel=sel):
                w, s0, s1 = st
                key = src[pl.ds(i * 16, 16)]
                valid = (i * 16 + lane) < cnt
                dig = lax.bitwise_and(
                    lax.shift_right_logical(key, sh), jnp.int32(255))
                m_gt = jnp.logical_and(dig > sel, valid)
                m_eq = jnp.logical_and(dig == sel, valid)
                s0, s1 = sum_fn(s0, s1, m_gt, _val_from_key(key))
                cs = plsc.cumsum(m_eq.astype(jnp.int32))
                plsc.store_scatter(dst, [cs + (w - 1)], key, mask=m_eq)
                return w + jnp.sum(m_eq.astype(jnp.int32)), s0, s1

            cnt, s0, s1 = lax.fori_loop(
                0, nch, compactn, (jnp.int32(0), s0, s1))
            t_key = lax.bitwise_or(t_key, lax.shift_left(sel, sh))
            src, dst = dst, src

        return c_gt, t_key, s0, s1

    def process_row(j, acc):
        row = wid * ROWS_PER_W + j

        pltpu.sync_copy(target_hbm.at[row], dbuf)

        def tsum(s0, s1, mask, v):
            return (s0 + jnp.where(mask, v, 0.0),
                    s1 + jnp.where(mask, v * v, 0.0))

        c_gt, t_key, s0, s1 = select_topk(False, tsum)
        ties = jnp.full((16,), K - c_gt, dtype=jnp.int32).astype(jnp.float32)
        t_v = _val_from_key(jnp.full((16,), t_key, dtype=jnp.int32))
        s1sum = _bsum(s0) + ties * t_v
        s2sum = _bsum(s1) + ties * t_v * t_v
        mu_v = s1sum * (1.0 / K)
        var_v = jnp.maximum(
            (s2sum - K * mu_v * mu_v) * (1.0 / (K - 1)), 0.0)
        beta_v = _sqrt_vec(var_v) * SQ6_PI + 1e-8
        invb_v = 1.0 / beta_v
        lb_v = _log_vec(beta_v)

        pltpu.sync_copy(pred_hbm.at[row], dbuf)

        def psum(s0, s1, mask, v):
            z = (v - mu_v) * invb_v
            e = jnp.exp(-z)
            return (s0 + jnp.where(mask, z, 0.0),
                    s1 + jnp.where(mask, e, 0.0))

        c_gt_p, t_key_p, sz, se = select_topk(True, psum)
        ties_p = jnp.full(
            (16,), K - c_gt_p, dtype=jnp.int32).astype(jnp.float32)
        tp_v = _val_from_key(jnp.full((16,), t_key_p, dtype=jnp.int32))
        z_t = (tp_v - mu_v) * invb_v
        zsum = _bsum(sz) + ties_p * z_t
        esum = _bsum(se) + ties_p * jnp.exp(-z_t)
        contrib = zsum + esum + float(K) * lb_v
        return acc + jnp.where(lane == 0, contrib, 0.0)

    acc = lax.fori_loop(0, ROWS_PER_W, process_row, zf)
    accbuf[...] = acc
    pltpu.sync_copy(accbuf, out_hbm.at[wid])


@jax.jit
def kernel(pred, target):
    mesh = plsc.VectorSubcoreMesh(
        core_axis_name="c", subcore_axis_name="s", num_cores=NC,
        num_subcores=NS)
    partials = pl.kernel(
        _body,
        out_type=jax.ShapeDtypeStruct((NW, 16), jnp.float32),
        mesh=mesh,
        compiler_params=pltpu.CompilerParams(needs_layout_passes=False),
        scratch_types=[
            pltpu.VMEM((N,), jnp.float32),
            pltpu.VMEM((N,), jnp.int32),
            pltpu.VMEM((N,), jnp.int32),
            pltpu.VMEM((4096,), jnp.int32),
            pltpu.VMEM((16,), jnp.float32),
        ],
    )(pred, target)
    return 0.1 * jnp.sum(partials) / (ROWS * K)


# speedup vs baseline: 7.6951x; 1.5337x over previous
import jax
import jax.numpy as jnp
import numpy as np
from jax import lax
from jax.experimental import pallas as pl
from jax.experimental.pallas import tpu as pltpu
from jax.experimental.pallas import tpu_sc as plsc

ROWS = 128
N = 32768
K = 1638
NC, NS = 2, 16
NW = NC * NS
ROWS_PER_W = ROWS // NW
NCH = N // 16
UNROLL = 8
LN2 = 0.6931471805599453
SQ6_PI = float(np.sqrt(6.0) / np.pi)
SQRT2 = 1.4142135623730951


def _key_from_val(v):
    b = plsc.bitcast(v, jnp.int32)
    m = lax.shift_right_arithmetic(b, 31)
    return lax.bitwise_xor(b, lax.bitwise_or(m, jnp.int32(-2147483648)))


def _val_from_key(key):
    m = lax.shift_right_arithmetic(key, 31)
    mask = lax.bitwise_or(lax.bitwise_not(m), jnp.int32(-2147483648))
    return plsc.bitcast(lax.bitwise_xor(key, mask), jnp.float32)


def _bsum(v):
    return jnp.full((16,), jnp.sum(v), dtype=v.dtype)


def _sqrt_vec(x):
    bits = plsc.bitcast(x, jnp.int32)
    g = plsc.bitcast(
        lax.shift_right_logical(bits, 1) + jnp.int32(0x1FBD1DF5), jnp.float32)
    for _ in range(4):
        g = 0.5 * (g + x / g)
    return g


def _log_vec(x):
    bits = plsc.bitcast(x, jnp.int32)
    e = lax.bitwise_and(lax.shift_right_logical(bits, 23), jnp.int32(255)) - 127
    m_bits = lax.bitwise_or(
        lax.bitwise_and(bits, jnp.int32(0x007FFFFF)), jnp.int32(0x3F800000))
    m = plsc.bitcast(m_bits, jnp.float32)
    big = m > SQRT2
    m = jnp.where(big, m * 0.5, m)
    e_f = (e + big.astype(jnp.int32)).astype(jnp.float32)
    s = (m - 1.0) / (m + 1.0)
    s2 = s * s
    p = 2.0 * s * (1.0 + s2 * (1.0 / 3.0 + s2 * (
        0.2 + s2 * (1.0 / 7.0 + s2 * (1.0 / 9.0)))))
    return e_f * LN2 + p


def _body(pred_hbm, target_hbm, out_hbm, dbuf, pbuf, cand, hist, accbuf,
          sem_t, sem_p):
    wid = lax.axis_index("s") * NC + lax.axis_index("c")
    lane = lax.broadcasted_iota(jnp.int32, (16,), 0)
    ones_i = jnp.full((16,), 1, dtype=jnp.int32)
    zeros_i = jnp.zeros((16,), dtype=jnp.int32)
    zf = jnp.zeros((16,), dtype=jnp.float32)

    def clear_hist():
        def cl(i, _):
            for j in range(16):
                hist[pl.ds(i * 256 + j * 16, 16)] = zeros_i
            return 0
        lax.fori_loop(0, 16, cl, 0)

    def scan_buckets(kk, g0):
        def gtot(g):
            acc = hist[pl.ds(g * 256, 16)]
            for j in range(1, 16):
                acc = acc + hist[pl.ds(g * 256 + j * 16, 16)]
            return jnp.sum(acc)

        def gcond(st):
            g, csum = st
            return csum + gtot(g) < kk

        def gbody(st):
            g, csum = st
            return g - 1, csum + gtot(g)

        g, csum = lax.while_loop(gcond, gbody, (g0, jnp.int32(0)))

        def tot(b):
            return jnp.sum(hist[pl.ds(b * 16, 16)])

        def bcond(st):
            b, csum = st
            return csum + tot(b) < kk

        def bbody(st):
            b, csum = st
            return b - 1, csum + tot(b)

        return lax.while_loop(bcond, bbody, (g * 16 + 15, csum))

    def select_topk(src, sum_fn):
        clear_hist()

        def hist1(i, vmax):
            for j in range(UNROLL):
                v = src[pl.ds((i * UNROLL + j) * 16, 16)]
                key = _key_from_val(v)
                dig = lax.shift_right_logical(key, 24)
                idx = lax.bitwise_or(lax.shift_left(dig, 4), lane)
                plsc.addupdate_scatter(hist, [idx], ones_i)
                vmax = jnp.maximum(vmax, v)
            return vmax

        vmax = lax.fori_loop(
            0, NCH // UNROLL, hist1, jnp.full((16,), -jnp.inf, jnp.float32))
        kmax = _key_from_val(jnp.full((16,), jnp.max(vmax), jnp.float32))
        g0 = jnp.max(lax.shift_right_logical(kmax, 28))
        sel, csum = scan_buckets(jnp.int32(K), g0)
        kk = K - csum
        c_gt = csum

        def compact1(i, st):
            c_v, s0, s1 = st
            for j in range(UNROLL):
                v = src[pl.ds((i * UNROLL + j) * 16, 16)]
                key = _key_from_val(v)
                dig = lax.shift_right_logical(key, 24)
                m_gt = dig > sel
                m_eq = dig == sel
                s0, s1 = sum_fn(s0, s1, m_gt, v)
                pos = lax.bitwise_or(lax.shift_left(c_v, 4), lane)
                plsc.store_scatter(cand, [pos], key, mask=m_eq)
                c_v = c_v + m_eq.astype(jnp.int32)
            return c_v, s0, s1

        c_v, s0, s1 = lax.fori_loop(
            0, NCH // UNROLL, compact1, (zeros_i, zf, zf))

        t_key = lax.shift_left(sel, 24)
        for sh in (16, 8, 0):
            clear_hist()
            nch = (jnp.max(c_v) + 3) // 4

            def histn(i, _, sh=sh, c_v=c_v):
                for j in range(4):
                    ch = i * 4 + j
                    key = cand[pl.ds(ch * 16, 16)]
                    valid = c_v > ch
                    dig = lax.bitwise_and(
                        lax.shift_right_logical(key, sh), jnp.int32(255))
                    idx = lax.bitwise_or(lax.shift_left(dig, 4), lane)
                    plsc.addupdate_scatter(hist, [idx], ones_i, mask=valid)
                return 0

            lax.fori_loop(0, nch, histn, 0)
            sel, csum = scan_buckets(kk, jnp.int32(15))
            kk = kk - csum
            c_gt = c_gt + csum

            def compactn(i, st, sh=sh, c_v=c_v, sel=sel):
                c2, s0, s1 = st
                for j in range(4):
                    ch = i * 4 + j
                    key = cand[pl.ds(ch * 16, 16)]
                    valid = c_v > ch
                    dig = lax.bitwise_and(
                        lax.shift_right_logical(key, sh), jnp.int32(255))
                    m_gt = jnp.logical_and(dig > sel, valid)
                    m_eq = jnp.logical_and(dig == sel, valid)
                    s0, s1 = sum_fn(s0, s1, m_gt, _val_from_key(key))
                    pos = lax.bitwise_or(lax.shift_left(c2, 4), lane)
                    plsc.store_scatter(cand, [pos], key, mask=m_eq)
                    c2 = c2 + m_eq.astype(jnp.int32)
                return c2, s0, s1

            c_v, s0, s1 = lax.fori_loop(0, nch, compactn, (zeros_i, s0, s1))
            t_key = lax.bitwise_or(t_key, lax.shift_left(sel, sh))

        return c_gt, t_key, s0, s1

    def dma_row(hbm, row, buf, sem):
        return pltpu.make_async_copy(hbm.at[row], buf, sem)

    def process_row(j, acc):
        row = wid * ROWS_PER_W + j
        nxt = jnp.minimum(row + 1, jnp.int32(ROWS - 1))

        dma_row(target_hbm, row, dbuf, sem_t).wait()

        def tsum(s0, s1, mask, v):
            return (s0 + jnp.where(mask, v, 0.0),
                    s1 + jnp.where(mask, v * v, 0.0))

        c_gt, t_key, s0, s1 = select_topk(dbuf, tsum)
        dma_row(target_hbm, nxt, dbuf, sem_t).start()

        ties = jnp.full((16,), K - c_gt, dtype=jnp.int32).astype(jnp.float32)
        t_v = _val_from_key(jnp.full((16,), t_key, dtype=jnp.int32))
        s1sum = _bsum(s0) + ties * t_v
        s2sum = _bsum(s1) + ties * t_v * t_v
        mu_v = s1sum * (1.0 / K)
        var_v = jnp.maximum(
            (s2sum - K * mu_v * mu_v) * (1.0 / (K - 1)), 0.0)
        beta_v = _sqrt_vec(var_v) * SQ6_PI + 1e-8
        invb_v = 1.0 / beta_v
        lb_v = _log_vec(beta_v)

        dma_row(pred_hbm, row, pbuf, sem_p).wait()

        def psum(s0, s1, mask, v):
            z = (v - mu_v) * invb_v
            e = jnp.exp(-z)
            return (s0 + jnp.where(mask, z, 0.0),
                    s1 + jnp.where(mask, e, 0.0))

        c_gt_p, t_key_p, sz, se = select_topk(pbuf, psum)
        dma_row(pred_hbm, nxt, pbuf, sem_p).start()

        ties_p = jnp.full(
            (16,), K - c_gt_p, dtype=jnp.int32).astype(jnp.float32)
        tp_v = _val_from_key(jnp.full((16,), t_key_p, dtype=jnp.int32))
        z_t = (tp_v - mu_v) * invb_v
        zsum = _bsum(sz) + ties_p * z_t
        esum = _bsum(se) + ties_p * jnp.exp(-z_t)
        contrib = zsum + esum + float(K) * lb_v
        return acc + jnp.where(lane == 0, contrib, 0.0)

    row0 = wid * ROWS_PER_W
    dma_row(target_hbm, row0, dbuf, sem_t).start()
    dma_row(pred_hbm, row0, pbuf, sem_p).start()
    acc = lax.fori_loop(0, ROWS_PER_W, process_row, zf)
    dma_row(target_hbm, row0, dbuf, sem_t).wait()
    dma_row(pred_hbm, row0, pbuf, sem_p).wait()
    accbuf[...] = acc
    pltpu.sync_copy(accbuf, out_hbm.at[wid])


@jax.jit
def kernel(pred, target):
    mesh = plsc.VectorSubcoreMesh(
        core_axis_name="c", subcore_axis_name="s", num_cores=NC,
        num_subcores=NS)
    partials = pl.kernel(
        _body,
        out_type=jax.ShapeDtypeStruct((NW, 16), jnp.float32),
        mesh=mesh,
        compiler_params=pltpu.CompilerParams(needs_layout_passes=False),
        scratch_types=[
            pltpu.VMEM((N,), jnp.float32),
            pltpu.VMEM((N,), jnp.float32),
            pltpu.VMEM((N,), jnp.int32),
            pltpu.VMEM((4096,), jnp.int32),
            pltpu.VMEM((16,), jnp.float32),
            pltpu.SemaphoreType.DMA,
            pltpu.SemaphoreType.DMA,
        ],
    )(pred, target)
    return 0.1 * jnp.sum(partials) / (ROWS * K)


# speedup vs baseline: 7.7242x; 1.0038x over previous
import jax
import jax.numpy as jnp
import numpy as np
from jax import lax
from jax.experimental import pallas as pl
from jax.experimental.pallas import tpu as pltpu
from jax.experimental.pallas import tpu_sc as plsc

ROWS = 128
N = 32768
K = 1638
NC, NS = 2, 16
NW = NC * NS
ROWS_PER_W = ROWS // NW
NCH = N // 16
UNROLL = 16
LN2 = 0.6931471805599453
SQ6_PI = float(np.sqrt(6.0) / np.pi)
SQRT2 = 1.4142135623730951


def _key_from_val(v):
    b = plsc.bitcast(v, jnp.int32)
    m = lax.shift_right_arithmetic(b, 31)
    return lax.bitwise_xor(b, lax.bitwise_or(m, jnp.int32(-2147483648)))


def _val_from_key(key):
    m = lax.shift_right_arithmetic(key, 31)
    mask = lax.bitwise_or(lax.bitwise_not(m), jnp.int32(-2147483648))
    return plsc.bitcast(lax.bitwise_xor(key, mask), jnp.float32)


def _bsum(v):
    return jnp.full((16,), jnp.sum(v), dtype=v.dtype)


def _sqrt_vec(x):
    bits = plsc.bitcast(x, jnp.int32)
    g = plsc.bitcast(
        lax.shift_right_logical(bits, 1) + jnp.int32(0x1FBD1DF5), jnp.float32)
    for _ in range(4):
        g = 0.5 * (g + x / g)
    return g


def _log_vec(x):
    bits = plsc.bitcast(x, jnp.int32)
    e = lax.bitwise_and(lax.shift_right_logical(bits, 23), jnp.int32(255)) - 127
    m_bits = lax.bitwise_or(
        lax.bitwise_and(bits, jnp.int32(0x007FFFFF)), jnp.int32(0x3F800000))
    m = plsc.bitcast(m_bits, jnp.float32)
    big = m > SQRT2
    m = jnp.where(big, m * 0.5, m)
    e_f = (e + big.astype(jnp.int32)).astype(jnp.float32)
    s = (m - 1.0) / (m + 1.0)
    s2 = s * s
    p = 2.0 * s * (1.0 + s2 * (1.0 / 3.0 + s2 * (
        0.2 + s2 * (1.0 / 7.0 + s2 * (1.0 / 9.0)))))
    return e_f * LN2 + p


def _body(pred_hbm, target_hbm, out_hbm, dbuf, pbuf, cand, hist, accbuf,
          sem_t, sem_p):
    wid = lax.axis_index("s") * NC + lax.axis_index("c")
    lane = lax.broadcasted_iota(jnp.int32, (16,), 0)
    ones_i = jnp.full((16,), 1, dtype=jnp.int32)
    zeros_i = jnp.zeros((16,), dtype=jnp.int32)
    zf = jnp.zeros((16,), dtype=jnp.float32)

    def clear_hist():
        def cl(i, _):
            for j in range(16):
                hist[pl.ds(i * 256 + j * 16, 16)] = zeros_i
            return 0
        lax.fori_loop(0, 16, cl, 0)

    def scan_buckets(kk, g0):
        def gtot(g):
            acc = hist[pl.ds(g * 256, 16)]
            for j in range(1, 16):
                acc = acc + hist[pl.ds(g * 256 + j * 16, 16)]
            return jnp.sum(acc)

        def gcond(st):
            g, csum = st
            return csum + gtot(g) < kk

        def gbody(st):
            g, csum = st
            return g - 1, csum + gtot(g)

        g, csum = lax.while_loop(gcond, gbody, (g0, jnp.int32(0)))

        def tot(b):
            return jnp.sum(hist[pl.ds(b * 16, 16)])

        def bcond(st):
            b, csum = st
            return csum + tot(b) < kk

        def bbody(st):
            b, csum = st
            return b - 1, csum + tot(b)

        return lax.while_loop(bcond, bbody, (g * 16 + 15, csum))

    def select_topk(src, sum_fn):
        clear_hist()

        def hist1(i, _):
            for j in range(UNROLL):
                v = src[pl.ds((i * UNROLL + j) * 16, 16)]
                key = _key_from_val(v)
                dig = lax.shift_right_logical(key, 24)
                idx = lax.bitwise_or(lax.shift_left(dig, 4), lane)
                plsc.addupdate_scatter(hist, [idx], ones_i)
            return 0

        lax.fori_loop(0, NCH // UNROLL, hist1, 0)
        sel, csum = scan_buckets(jnp.int32(K), jnp.int32(15))
        kk = K - csum
        c_gt = csum

        def compact1(i, st):
            c_v, s0, s1 = st
            for j in range(UNROLL):
                v = src[pl.ds((i * UNROLL + j) * 16, 16)]
                key = _key_from_val(v)
                dig = lax.shift_right_logical(key, 24)
                m_gt = dig > sel
                m_eq = dig == sel
                s0, s1 = sum_fn(s0, s1, m_gt, v)
                pos = lax.bitwise_or(lax.shift_left(c_v, 4), lane)
                plsc.store_scatter(cand, [pos], key, mask=m_eq)
                c_v = c_v + m_eq.astype(jnp.int32)
            return c_v, s0, s1

        c_v, s0, s1 = lax.fori_loop(
            0, NCH // UNROLL, compact1, (zeros_i, zf, zf))

        t_key = lax.shift_left(sel, 24)
        for sh in (16, 8, 0):
            clear_hist()
            nch = (jnp.max(c_v) + 3) // 4

            def histn(i, _, sh=sh, c_v=c_v):
                for j in range(4):
                    ch = i * 4 + j
                    key = cand[pl.ds(ch * 16, 16)]
                    valid = c_v > ch
                    dig = lax.bitwise_and(
                        lax.shift_right_logical(key, sh), jnp.int32(255))
                    idx = lax.bitwise_or(lax.shift_left(dig, 4), lane)
                    plsc.addupdate_scatter(hist, [idx], ones_i, mask=valid)
                return 0

            lax.fori_loop(0, nch, histn, 0)
            sel, csum = scan_buckets(kk, jnp.int32(15))
            kk = kk - csum
            c_gt = c_gt + csum

            def compactn(i, st, sh=sh, c_v=c_v, sel=sel):
                c2, s0, s1 = st
                for j in range(4):
                    ch = i * 4 + j
                    key = cand[pl.ds(ch * 16, 16)]
                    valid = c_v > ch
                    dig = lax.bitwise_and(
                        lax.shift_right_logical(key, sh), jnp.int32(255))
                    m_gt = jnp.logical_and(dig > sel, valid)
                    m_eq = jnp.logical_and(dig == sel, valid)
                    s0, s1 = sum_fn(s0, s1, m_gt, _val_from_key(key))
                    pos = lax.bitwise_or(lax.shift_left(c2, 4), lane)
                    plsc.store_scatter(cand, [pos], key, mask=m_eq)
                    c2 = c2 + m_eq.astype(jnp.int32)
                return c2, s0, s1

            c_v, s0, s1 = lax.fori_loop(0, nch, compactn, (zeros_i, s0, s1))
            t_key = lax.bitwise_or(t_key, lax.shift_left(sel, sh))

        return c_gt, t_key, s0, s1

    def dma_row(hbm, row, buf, sem):
        return pltpu.make_async_copy(hbm.at[row], buf, sem)

    def process_row(j, acc):
        row = wid * ROWS_PER_W + j
        nxt = jnp.minimum(row + 1, jnp.int32(ROWS - 1))

        dma_row(target_hbm, row, dbuf, sem_t).wait()

        def tsum(s0, s1, mask, v):
            return (s0 + jnp.where(mask, v, 0.0),
                    s1 + jnp.where(mask, v * v, 0.0))

        c_gt, t_key, s0, s1 = select_topk(dbuf, tsum)
        dma_row(target_hbm, nxt, dbuf, sem_t).start()

        ties = jnp.full((16,), K - c_gt, dtype=jnp.int32).astype(jnp.float32)
        t_v = _val_from_key(jnp.full((16,), t_key, dtype=jnp.int32))
        s1sum = _bsum(s0) + ties * t_v
        s2sum = _bsum(s1) + ties * t_v * t_v
        mu_v = s1sum * (1.0 / K)
        var_v = jnp.maximum(
            (s2sum - K * mu_v * mu_v) * (1.0 / (K - 1)), 0.0)
        beta_v = _sqrt_vec(var_v) * SQ6_PI + 1e-8
        invb_v = 1.0 / beta_v
        lb_v = _log_vec(beta_v)

        dma_row(pred_hbm, row, pbuf, sem_p).wait()

        def psum(s0, s1, mask, v):
            z = (v - mu_v) * invb_v
            e = jnp.exp(-z)
            return (s0 + jnp.where(mask, z, 0.0),
                    s1 + jnp.where(mask, e, 0.0))

        c_gt_p, t_key_p, sz, se = select_topk(pbuf, psum)
        dma_row(pred_hbm, nxt, pbuf, sem_p).start()

        ties_p = jnp.full(
            (16,), K - c_gt_p, dtype=jnp.int32).astype(jnp.float32)
        tp_v = _val_from_key(jnp.full((16,), t_key_p, dtype=jnp.int32))
        z_t = (tp_v - mu_v) * invb_v
        zsum = _bsum(sz) + ties_p * z_t
        esum = _bsum(se) + ties_p * jnp.exp(-z_t)
        contrib = zsum + esum + float(K) * lb_v
        return acc + jnp.where(lane == 0, contrib, 0.0)

    row0 = wid * ROWS_PER_W
    dma_row(target_hbm, row0, dbuf, sem_t).start()
    dma_row(pred_hbm, row0, pbuf, sem_p).start()
    acc = lax.fori_loop(0, ROWS_PER_W, process_row, zf)
    dma_row(target_hbm, row0, dbuf, sem_t).wait()
    dma_row(pred_hbm, row0, pbuf, sem_p).wait()
    accbuf[...] = acc
    pltpu.sync_copy(accbuf, out_hbm.at[wid])


@jax.jit
def kernel(pred, target):
    mesh = plsc.VectorSubcoreMesh(
        core_axis_name="c", subcore_axis_name="s", num_cores=NC,
        num_subcores=NS)
    partials = pl.kernel(
        _body,
        out_type=jax.ShapeDtypeStruct((NW, 16), jnp.float32),
        mesh=mesh,
        compiler_params=pltpu.CompilerParams(needs_layout_passes=False),
        scratch_types=[
            pltpu.VMEM((N,), jnp.float32),
            pltpu.VMEM((N,), jnp.float32),
            pltpu.VMEM((N,), jnp.int32),
            pltpu.VMEM((4096,), jnp.int32),
            pltpu.VMEM((16,), jnp.float32),
            pltpu.SemaphoreType.DMA,
            pltpu.SemaphoreType.DMA,
        ],
    )(pred, target)
    return 0.1 * jnp.sum(partials) / (ROWS * K)


# speedup vs baseline: 15.2053x; 1.9685x over previous
import jax
import jax.numpy as jnp
import numpy as np
from jax import lax
from jax.experimental import pallas as pl
from jax.experimental.pallas import tpu as pltpu
from jax.experimental.pallas import tpu_sc as plsc

ROWS = 128
N = 32768
K = 1638
NC, NS = 2, 16
NW = NC * NS
ROWS_PER_W = ROWS // NW
NCH = N // 16
UNROLL = 16
LN2 = 0.6931471805599453
SQ6_PI = float(np.sqrt(6.0) / np.pi)
SQRT2 = 1.4142135623730951


def _key_from_val(v):
    b = plsc.bitcast(v, jnp.int32)
    m = lax.shift_right_arithmetic(b, 31)
    return lax.bitwise_xor(b, lax.bitwise_or(m, jnp.int32(-2147483648)))


def _val_from_key(key):
    m = lax.shift_right_arithmetic(key, 31)
    mask = lax.bitwise_or(lax.bitwise_not(m), jnp.int32(-2147483648))
    return plsc.bitcast(lax.bitwise_xor(key, mask), jnp.float32)


def _tree_sum(xs):
    xs = list(xs)
    while len(xs) > 1:
        nxt = [xs[i] + xs[i + 1] for i in range(0, len(xs) - 1, 2)]
        if len(xs) % 2:
            nxt.append(xs[-1])
        xs = nxt
    return xs[0]


def _bsum(v):
    return jnp.full((16,), jnp.sum(v), dtype=v.dtype)


def _sqrt_vec(x):
    bits = plsc.bitcast(x, jnp.int32)
    g = plsc.bitcast(
        lax.shift_right_logical(bits, 1) + jnp.int32(0x1FBD1DF5), jnp.float32)
    for _ in range(4):
        g = 0.5 * (g + x / g)
    return g


def _log_vec(x):
    bits = plsc.bitcast(x, jnp.int32)
    e = lax.bitwise_and(lax.shift_right_logical(bits, 23), jnp.int32(255)) - 127
    m_bits = lax.bitwise_or(
        lax.bitwise_and(bits, jnp.int32(0x007FFFFF)), jnp.int32(0x3F800000))
    m = plsc.bitcast(m_bits, jnp.float32)
    big = m > SQRT2
    m = jnp.where(big, m * 0.5, m)
    e_f = (e + big.astype(jnp.int32)).astype(jnp.float32)
    s = (m - 1.0) / (m + 1.0)
    s2 = s * s
    p = 2.0 * s * (1.0 + s2 * (1.0 / 3.0 + s2 * (
        0.2 + s2 * (1.0 / 7.0 + s2 * (1.0 / 9.0)))))
    return e_f * LN2 + p


def _body(pred_hbm, target_hbm, out_hbm, dbuf, pbuf, cand, hist, accbuf,
          sem_t, sem_p):
    wid = lax.axis_index("s") * NC + lax.axis_index("c")
    lane = lax.broadcasted_iota(jnp.int32, (16,), 0)
    ones_i = jnp.full((16,), 1, dtype=jnp.int32)
    zeros_i = jnp.zeros((16,), dtype=jnp.int32)
    zf = jnp.zeros((16,), dtype=jnp.float32)

    def clear_hist():
        def cl(i, _):
            for j in range(16):
                hist[pl.ds(i * 256 + j * 16, 16)] = zeros_i
            return 0
        lax.fori_loop(0, 16, cl, 0)

    def scan_buckets(kk, g0):
        def gtot(g):
            acc = hist[pl.ds(g * 256, 16)]
            for j in range(1, 16):
                acc = acc + hist[pl.ds(g * 256 + j * 16, 16)]
            return jnp.sum(acc)

        def gcond(st):
            g, csum = st
            return csum + gtot(g) < kk

        def gbody(st):
            g, csum = st
            return g - 1, csum + gtot(g)

        g, csum = lax.while_loop(gcond, gbody, (g0, jnp.int32(0)))

        def tot(b):
            return jnp.sum(hist[pl.ds(b * 16, 16)])

        def bcond(st):
            b, csum = st
            return csum + tot(b) < kk

        def bbody(st):
            b, csum = st
            return b - 1, csum + tot(b)

        return lax.while_loop(bcond, bbody, (g * 16 + 15, csum))

    def select_topk(src, sum_fn):
        clear_hist()

        def hist1(i, _):
            vs = [src[pl.ds((i * UNROLL + j) * 16, 16)]
                  for j in range(UNROLL)]
            keys = [_key_from_val(v) for v in vs]
            idxs = [lax.bitwise_or(
                lax.shift_left(lax.shift_right_logical(k, 24), 4), lane)
                for k in keys]
            for idx in idxs:
                plsc.addupdate_scatter(hist, [idx], ones_i)
            return 0

        lax.fori_loop(0, NCH // UNROLL, hist1, 0)
        sel, csum = scan_buckets(jnp.int32(K), jnp.int32(15))
        kk = K - csum
        c_gt = csum

        def compact1(i, st):
            c_v, s0, s1 = st
            vs = [src[pl.ds((i * UNROLL + j) * 16, 16)]
                  for j in range(UNROLL)]
            keys = [_key_from_val(v) for v in vs]
            digs = [lax.shift_right_logical(k, 24) for k in keys]
            m_gts = [d > sel for d in digs]
            m_eqs = [d == sel for d in digs]
            cons = [sum_fn(m, v) for m, v in zip(m_gts, vs)]
            s0 = s0 + _tree_sum([c[0] for c in cons])
            s1 = s1 + _tree_sum([c[1] for c in cons])
            for j in range(UNROLL):
                pos = lax.bitwise_or(lax.shift_left(c_v, 4), lane)
                plsc.store_scatter(cand, [pos], keys[j], mask=m_eqs[j])
                c_v = c_v + m_eqs[j].astype(jnp.int32)
            return c_v, s0, s1

        c_v, s0, s1 = lax.fori_loop(
            0, NCH // UNROLL, compact1, (zeros_i, zf, zf))

        t_key = lax.shift_left(sel, 24)
        for sh in (16, 8, 0):
            clear_hist()
            nch = (jnp.max(c_v) + 3) // 4

            def histn(i, _, sh=sh, c_v=c_v):
                for j in range(4):
                    ch = i * 4 + j
                    key = cand[pl.ds(ch * 16, 16)]
                    valid = c_v > ch
                    dig = lax.bitwise_and(
                        lax.shift_right_logical(key, sh), jnp.int32(255))
                    idx = lax.bitwise_or(lax.shift_left(dig, 4), lane)
                    plsc.addupdate_scatter(hist, [idx], ones_i, mask=valid)
                return 0

            lax.fori_loop(0, nch, histn, 0)
            sel, csum = scan_buckets(kk, jnp.int32(15))
            kk = kk - csum
            c_gt = c_gt + csum

            def compactn(i, st, sh=sh, c_v=c_v, sel=sel):
                c2, s0, s1 = st
                for j in range(4):
                    ch = i * 4 + j
                    key = cand[pl.ds(ch * 16, 16)]
                    valid = c_v > ch
                    dig = lax.bitwise_and(
                        lax.shift_right_logical(key, sh), jnp.int32(255))
                    m_gt = jnp.logical_and(dig > sel, valid)
                    m_eq = jnp.logical_and(dig == sel, valid)
                    c0, c1 = sum_fn(m_gt, _val_from_key(key))
                    s0, s1 = s0 + c0, s1 + c1
                    pos = lax.bitwise_or(lax.shift_left(c2, 4), lane)
                    plsc.store_scatter(cand, [pos], key, mask=m_eq)
                    c2 = c2 + m_eq.astype(jnp.int32)
                return c2, s0, s1

            c_v, s0, s1 = lax.fori_loop(0, nch, compactn, (zeros_i, s0, s1))
            t_key = lax.bitwise_or(t_key, lax.shift_left(sel, sh))

        return c_gt, t_key, s0, s1

    def dma_row(hbm, row, buf, sem):
        return pltpu.make_async_copy(hbm.at[row], buf, sem)

    def process_row(j, acc):
        row = wid * ROWS_PER_W + j
        nxt = jnp.minimum(row + 1, jnp.int32(ROWS - 1))

        dma_row(target_hbm, row, dbuf, sem_t).wait()

        def tsum(mask, v):
            return jnp.where(mask, v, 0.0), jnp.where(mask, v * v, 0.0)

        c_gt, t_key, s0, s1 = select_topk(dbuf, tsum)
        dma_row(target_hbm, nxt, dbuf, sem_t).start()

        ties = jnp.full((16,), K - c_gt, dtype=jnp.int32).astype(jnp.float32)
        t_v = _val_from_key(jnp.full((16,), t_key, dtype=jnp.int32))
        s1sum = _bsum(s0) + ties * t_v
        s2sum = _bsum(s1) + ties * t_v * t_v
        mu_v = s1sum * (1.0 / K)
        var_v = jnp.maximum(
            (s2sum - K * mu_v * mu_v) * (1.0 / (K - 1)), 0.0)
        beta_v = _sqrt_vec(var_v) * SQ6_PI + 1e-8
        invb_v = 1.0 / beta_v
        lb_v = _log_vec(beta_v)

        dma_row(pred_hbm, row, pbuf, sem_p).wait()

        def psum(mask, v):
            z = (v - mu_v) * invb_v
            e = jnp.exp(-z)
            return jnp.where(mask, z, 0.0), jnp.where(mask, e, 0.0)

        c_gt_p, t_key_p, sz, se = select_topk(pbuf, psum)
        dma_row(pred_hbm, nxt, pbuf, sem_p).start()

        ties_p = jnp.full(
            (16,), K - c_gt_p, dtype=jnp.int32).astype(jnp.float32)
        tp_v = _val_from_key(jnp.full((16,), t_key_p, dtype=jnp.int32))
        z_t = (tp_v - mu_v) * invb_v
        zsum = _bsum(sz) + ties_p * z_t
        esum = _bsum(se) + ties_p * jnp.exp(-z_t)
        contrib = zsum + esum + float(K) * lb_v
        return acc + jnp.where(lane == 0, contrib, 0.0)

    row0 = wid * ROWS_PER_W
    dma_row(target_hbm, row0, dbuf, sem_t).start()
    dma_row(pred_hbm, row0, pbuf, sem_p).start()
    acc = lax.fori_loop(0, ROWS_PER_W, process_row, zf)
    dma_row(target_hbm, row0, dbuf, sem_t).wait()
    dma_row(pred_hbm, row0, pbuf, sem_p).wait()
    accbuf[...] = acc
    pltpu.sync_copy(accbuf, out_hbm.at[wid])


@jax.jit
def kernel(pred, target):
    mesh = plsc.VectorSubcoreMesh(
        core_axis_name="c", subcore_axis_name="s", num_cores=NC,
        num_subcores=NS)
    partials = pl.kernel(
        _body,
        out_type=jax.ShapeDtypeStruct((NW, 16), jnp.float32),
        mesh=mesh,
        compiler_params=pltpu.CompilerParams(needs_layout_passes=False),
        scratch_types=[
            pltpu.VMEM((N,), jnp.float32),
            pltpu.VMEM((N,), jnp.float32),
            pltpu.VMEM((N,), jnp.int32),
            pltpu.VMEM((4096,), jnp.int32),
            pltpu.VMEM((16,), jnp.float32),
            pltpu.SemaphoreType.DMA,
            pltpu.SemaphoreType.DMA,
        ],
    )(pred, target)
    return 0.1 * jnp.sum(partials) / (ROWS * K)
